# baseline (speedup 1.0000x reference)
import functools

import jax
import jax.numpy as jnp
from jax import lax
from jax.experimental import pallas as pl
from jax.experimental.pallas import tpu as pltpu
from jax.experimental.pallas import tpu_sc as plsc

_K = 4096
_B = 1048576
_R, _C = 32, 128

_NC = 2
_NS = 16
_L = 16
_NW = _NC * _NS
_S = _B // _NW


def _counts_body(sym_ref, lam_ref, cum_ref):
    lam = lam_ref[0, 0]
    re = sym_ref[0]
    im = sym_ref[1]
    abs2 = re * re + im * im
    mb = jnp.exp(-lam * abs2)
    p = mb / jnp.sum(mb)
    scaled = p * jnp.float32(_B)
    fl = jnp.floor(scaled)
    frac = scaled - fl
    rem = jnp.float32(_B) - jnp.sum(fl)

    fq = lax.bitcast_convert_type(frac, jnp.int32)
    def _step(_, lh):
        lo, hi = lh
        mid = (lo + hi) >> 1
        cnt = jnp.sum((fq > mid).astype(jnp.float32))
        pred = cnt < rem
        return jnp.where(pred, lo, mid + 1), jnp.where(pred, mid, hi)

    _, t = lax.fori_loop(0, 31, _step, (jnp.int32(0), jnp.int32(0x3F800000)))

    gt = fq > t
    eq = fq == t
    n_gt = jnp.sum(gt.astype(jnp.float32))
    tie_need = rem - n_gt

    br = lax.broadcasted_iota(jnp.int32, (_C, _C), 0)
    bc = lax.broadcasted_iota(jnp.int32, (_C, _C), 1)
    t_incl = (br <= bc).astype(jnp.float32)
    ar = lax.broadcasted_iota(jnp.int32, (_R, _R), 0)
    ac = lax.broadcasted_iota(jnp.int32, (_R, _R), 1)
    s_strict = (ac < ar).astype(jnp.float32)

    def _cumsum_incl(x):
        row_incl = jnp.dot(x, t_incl, precision=lax.Precision.HIGHEST)
        rt = jnp.sum(x, axis=1, keepdims=True)
        pref = jnp.dot(s_strict, rt, precision=lax.Precision.HIGHEST)
        return row_incl + pref

    eqf = eq.astype(jnp.float32)
    eq_excl = _cumsum_incl(eqf) - eqf
    bonus = gt.astype(jnp.float32) + jnp.where(
        eq & (eq_excl < tie_need), 1.0, 0.0)
    counts = fl + bonus
    cum_ref[...] = _cumsum_incl(counts).astype(jnp.int32)


_counts_call = pl.pallas_call(
    _counts_body,
    out_shape=jax.ShapeDtypeStruct((_R, _C), jnp.int32),
    in_specs=[
        pl.BlockSpec(memory_space=pltpu.VMEM),
        pl.BlockSpec(memory_space=pltpu.SMEM),
    ],
    out_specs=pl.BlockSpec(memory_space=pltpu.VMEM),
)


_NG = _S // _L
_GB = 512
_NB = _S // _GB
_KP = _K + _L


def _expand_body(cum_hbm, out_hbm, cum_v, b_v, out_v, sem):
    wid = lax.axis_index("s") * _NC + lax.axis_index("c")
    j0 = wid * _S
    pltpu.sync_copy(cum_hbm, cum_v.at[pl.ds(0, _K)])
    cum_v[pl.ds(_K, _L)] = jnp.full((_L,), _B, jnp.int32)

    zeros = jnp.zeros((_L,), jnp.int32)
    full_k = jnp.full((_L,), _K, jnp.int32)
    iota = lax.iota(jnp.int32, _L)

    def _bsearch(j, lo, hi):
        for _ in range(13):
            mid = (lo + hi) >> 1
            v = plsc.load_gather(cum_v, [mid])
            le = v <= j
            lo = jnp.where(le, mid + 1, lo)
            hi = jnp.where(le, hi, mid)
        return lo

    for t in range(_NB // _L):
        j_b = j0 + (t * _L + iota) * _GB
        b_v[pl.ds(t * _L, _L)] = _bsearch(j_b, zeros, full_k)
    j_tail = j0 + (_NB - 15 + iota) * _GB
    b_v[pl.ds(_NB - 15, _L)] = _bsearch(j_tail, zeros, full_k)

    def _p2(mb, carry):
        v2 = b_v[pl.ds(mb, _L)]
        s0 = v2[0]
        s1 = v2[1]
        base = mb * _GB
        s0v = jnp.broadcast_to(s0, (_L,))

        @pl.when(s1 - s0 <= 4)
        def _():
            cv = cum_v[pl.ds(s0, _L)]
            c0 = jnp.broadcast_to(cv[0], (_L,))
            c1 = jnp.broadcast_to(cv[1], (_L,))
            c2 = jnp.broadcast_to(cv[2], (_L,))
            c3 = jnp.broadcast_to(cv[3], (_L,))
            for g in range(_GB // _L):
                j = j0 + base + g * _L + iota
                out_v[pl.ds(base + g * _L, _L)] = (
                    s0v + (c0 <= j).astype(jnp.int32)
                    + (c1 <= j).astype(jnp.int32)
                    + (c2 <= j).astype(jnp.int32)
                    + (c3 <= j).astype(jnp.int32))

        @pl.when(s1 - s0 > 4)
        def _():
            s1v = jnp.broadcast_to(s1, (_L,))

            def _grp(g, c):
                j = j0 + base + g * _L + iota
                out_v[pl.ds(base + g * _L, _L)] = _bsearch(j, s0v, s1v)
                return c

            lax.fori_loop(0, _GB // _L, _grp, 0)

        return carry

    _Q = _S // 4
    handles = []
    for q in range(4):
        lax.fori_loop(q * (_NB // 4), (q + 1) * (_NB // 4), _p2, 0)
        handles.append(pltpu.async_copy(
            out_v.at[pl.ds(q * _Q, _Q)],
            out_hbm.at[pl.ds(j0 + q * _Q, _Q)],
            sem))
    for h in handles:
        h.wait()


_expand_call = pl.kernel(
    _expand_body,
    out_type=jax.ShapeDtypeStruct((_B,), jnp.int32),
    mesh=plsc.VectorSubcoreMesh(
        core_axis_name="c", subcore_axis_name="s",
        num_cores=_NC, num_subcores=_NS),
    scratch_types=[
        pltpu.VMEM((_KP,), jnp.int32),
        pltpu.VMEM((_NB + 1 + 15,), jnp.int32),
        pltpu.VMEM((_S,), jnp.int32),
        pltpu.SemaphoreType.DMA,
    ],
    compiler_params=pltpu.CompilerParams(
        needs_layout_passes=False,
        use_tc_tiling_on_sc=False,
    ),
)


def kernel(symbols, logits, batchsize):
    del batchsize
    sym = symbols.T.reshape(2, _R, _C)
    lam = logits.reshape(1, 1)
    cum = _counts_call(sym, lam).reshape(_K)
    return _expand_call(cum)
